# Initial kernel scaffold; baseline (speedup 1.0000x reference)
import jax
import jax.numpy as jnp
from jax.experimental import pallas as pl


def kernel(input_node_indices, node_features, edge_index, params):
    raise NotImplementedError("write your pallas kernel here")


# speedup vs baseline: 7.3459x; 7.3459x over previous
import functools

import jax
import jax.numpy as jnp
from jax import lax
from jax.experimental import pallas as pl
from jax.experimental.pallas import tpu as pltpu
from jax.experimental.pallas import tpu_sc as plsc

N = 10000
E = 320000
D_FEAT = 128
H = 32
NUM_CLASSES = 16
B = 1024

NC = 2
NS = 16
NW = NC * NS
EPW = E // NW
CH = 128
NFULL = EPW // CH
TAIL = EPW - NFULL * CH
NPAD = 10240
RP_SC = NPAD // NS


def _fold_ffn(p):
    s = p["g"] / jnp.sqrt(p["v"] + 1e-3)
    t = p["b"] - p["m"] * s
    W = p["W"] * s[:, None]
    b = t @ p["W"] + p["bias"]
    return W, b


def _stage_a_body(nf, w_pre, b_pre, w_p1, b_p1, xpre_o, m1_o):
    x = jax.nn.gelu(jnp.dot(nf[...], w_pre[...],
                            preferred_element_type=jnp.float32, precision=lax.Precision.HIGHEST) + b_pre[...])
    xpre_o[...] = x
    m1_o[...] = jax.nn.gelu(jnp.dot(x, w_p1[...],
                                    preferred_element_type=jnp.float32, precision=lax.Precision.HIGHEST) + b_p1[...])


def _stage_b_body(xpre, aggp, cntp, w_u1a, w_u1b, b_u1, w_p2, b_p2,
                  x1_o, m2_o):
    cnt = cntp[0, :N, 0:1] + cntp[1, :N, 0:1]
    inv = 1.0 / jnp.maximum(cnt, 1.0)
    agg = (aggp[0, :N] + aggp[1, :N]) * inv
    x = xpre[...]
    h = jax.nn.gelu(jnp.dot(x, w_u1a[...], preferred_element_type=jnp.float32, precision=lax.Precision.HIGHEST)
                    + jnp.dot(agg, w_u1b[...], preferred_element_type=jnp.float32, precision=lax.Precision.HIGHEST)
                    + b_u1[...])
    x1 = h + x
    x1_o[...] = x1
    m2_o[...] = jax.nn.gelu(jnp.dot(x1, w_p2[...],
                                    preferred_element_type=jnp.float32, precision=lax.Precision.HIGHEST) + b_p2[...])


def _stage_c_body(x1, aggp, cntp, w_u2a, w_u2b, b_u2, w_post, b_post,
                  w_out, b_out, out_o):
    cnt = cntp[0, :N, 0:1] + cntp[1, :N, 0:1]
    inv = 1.0 / jnp.maximum(cnt, 1.0)
    agg = (aggp[0, :N] + aggp[1, :N]) * inv
    x = x1[...]
    h = jax.nn.gelu(jnp.dot(x, w_u2a[...], preferred_element_type=jnp.float32, precision=lax.Precision.HIGHEST)
                    + jnp.dot(agg, w_u2b[...], preferred_element_type=jnp.float32, precision=lax.Precision.HIGHEST)
                    + b_u2[...])
    x2 = h + x
    xp = jax.nn.gelu(jnp.dot(x2, w_post[...],
                             preferred_element_type=jnp.float32, precision=lax.Precision.HIGHEST) + b_post[...])
    out_o[...] = jnp.dot(xp, w_out[...],
                         preferred_element_type=jnp.float32, precision=lax.Precision.HIGHEST) + b_out[...]


def _edge_pass_body(with_counts, m_hbm, dst_hbm, nbr_hbm, zeros32_hbm,
                    zeros16_hbm, ones_hbm, agg_out, cnt_out,
                    nbr_v, dst_v, rows_v, ones_v, nbr_t, dst_t, rows_t,
                    agg_sh, cnt_sh):
    cid = lax.axis_index("c")
    sid = lax.axis_index("s")
    wid = cid * NS + sid
    base = pl.multiple_of(wid * EPW, 8)

    rbase = pl.multiple_of(sid * RP_SC, 8)
    pltpu.sync_copy(zeros32_hbm.at[pl.ds(rbase, RP_SC)],
                    agg_sh.at[pl.ds(rbase, RP_SC)])
    if with_counts:
        pltpu.sync_copy(zeros16_hbm.at[pl.ds(rbase, RP_SC)],
                        cnt_sh.at[pl.ds(rbase, RP_SC)])
        pltpu.sync_copy(ones_hbm, ones_v)
    plsc.subcore_barrier()

    def chunk(j, _):
        off = pl.multiple_of(base + j * CH, 8)
        pltpu.sync_copy(nbr_hbm.at[pl.ds(off, CH)], nbr_v)
        pltpu.sync_copy(dst_hbm.at[pl.ds(off, CH)], dst_v)
        pltpu.sync_copy(m_hbm.at[nbr_v], rows_v)
        pltpu.sync_copy(rows_v, agg_sh.at[dst_v], add=True)
        if with_counts:
            pltpu.sync_copy(ones_v, cnt_sh.at[dst_v], add=True)
        return 0

    lax.fori_loop(0, NFULL, chunk, 0)

    toff = pl.multiple_of(base + NFULL * CH, 8)
    pltpu.sync_copy(nbr_hbm.at[pl.ds(toff, TAIL)], nbr_t)
    pltpu.sync_copy(dst_hbm.at[pl.ds(toff, TAIL)], dst_t)
    pltpu.sync_copy(m_hbm.at[nbr_t], rows_t)
    pltpu.sync_copy(rows_t, agg_sh.at[dst_t], add=True)
    if with_counts:
        pltpu.sync_copy(ones_v.at[pl.ds(0, TAIL)], cnt_sh.at[dst_t], add=True)

    plsc.subcore_barrier()

    pltpu.sync_copy(agg_sh.at[pl.ds(rbase, RP_SC)],
                    agg_out.at[cid, pl.ds(rbase, RP_SC)])
    if with_counts:
        pltpu.sync_copy(cnt_sh.at[pl.ds(rbase, RP_SC)],
                        cnt_out.at[cid, pl.ds(rbase, RP_SC)])


def _make_edge_pass(with_counts):
    mesh = plsc.VectorSubcoreMesh(core_axis_name="c", subcore_axis_name="s")
    out_type = [jax.ShapeDtypeStruct((NC, NPAD, H), jnp.float32)]
    if with_counts:
        out_type.append(jax.ShapeDtypeStruct((NC, NPAD, 16), jnp.float32))
    scratch = [
        pltpu.VMEM((CH,), jnp.int32),
        pltpu.VMEM((CH,), jnp.int32),
        pltpu.VMEM((CH, H), jnp.float32),
        pltpu.VMEM((CH, 16), jnp.float32),
        pltpu.VMEM((TAIL,), jnp.int32),
        pltpu.VMEM((TAIL,), jnp.int32),
        pltpu.VMEM((TAIL, H), jnp.float32),
        pltpu.VMEM_SHARED((NPAD, H), jnp.float32),
        pltpu.VMEM_SHARED((NPAD, 16), jnp.float32),
    ]

    if with_counts:
        def body(m, d, nb, z32, z16, on, agg_o, cnt_o, *s):
            _edge_pass_body(True, m, d, nb, z32, z16, on, agg_o, cnt_o, *s)
    else:
        def body(m, d, nb, z32, z16, on, agg_o, *s):
            _edge_pass_body(False, m, d, nb, z32, z16, on, agg_o, None, *s)

    return pl.kernel(body, out_type=out_type, mesh=mesh, scratch_types=scratch,
                     compiler_params=pltpu.CompilerParams(use_tc_tiling_on_sc=False))


BPW = B // NW


def _final_gather_body(table_hbm, idx_hbm, out_hbm, idx_v, rows_v):
    wid = lax.axis_index("c") * NS + lax.axis_index("s")
    base = pl.multiple_of(wid * BPW, 8)
    pltpu.sync_copy(idx_hbm.at[pl.ds(base, BPW)], idx_v)
    pltpu.sync_copy(table_hbm.at[idx_v], rows_v)
    pltpu.sync_copy(rows_v, out_hbm.at[pl.ds(base, BPW)])


def _make_final_gather():
    mesh = plsc.VectorSubcoreMesh(core_axis_name="c", subcore_axis_name="s")
    return pl.kernel(
        _final_gather_body,
        out_type=jax.ShapeDtypeStruct((B, NUM_CLASSES), jnp.float32),
        mesh=mesh,
        scratch_types=[
            pltpu.VMEM((BPW,), jnp.int32),
            pltpu.VMEM((BPW, NUM_CLASSES), jnp.float32),
        ],
        compiler_params=pltpu.CompilerParams(use_tc_tiling_on_sc=False),
    )


def kernel(input_node_indices, node_features, edge_index, params):
    dst = edge_index[0]
    nbr = edge_index[1]

    w_pre, b_pre = _fold_ffn(params["pre"])
    w_p1, b_p1 = _fold_ffn(params["c1_prep"])
    w_u1, b_u1 = _fold_ffn(params["c1_upd"])
    w_p2, b_p2 = _fold_ffn(params["c2_prep"])
    w_u2, b_u2 = _fold_ffn(params["c2_upd"])
    w_post, b_post = _fold_ffn(params["post"])
    w_u1a, w_u1b = w_u1[:H], w_u1[H:]
    w_u2a, w_u2b = w_u2[:H], w_u2[H:]

    zeros32 = jnp.zeros((NPAD, H), jnp.float32)
    zeros16 = jnp.zeros((NPAD, 16), jnp.float32)
    ones = jnp.ones((CH, 16), jnp.float32)

    xpre, m1 = pl.pallas_call(
        _stage_a_body,
        out_shape=[jax.ShapeDtypeStruct((N, H), jnp.float32),
                   jax.ShapeDtypeStruct((N, H), jnp.float32)],
    )(node_features, w_pre, b_pre, w_p1, b_p1)

    edge_pass1 = _make_edge_pass(True)
    aggp1, cntp = edge_pass1(m1, dst, nbr, zeros32, zeros16, ones)

    x1, m2 = pl.pallas_call(
        _stage_b_body,
        out_shape=[jax.ShapeDtypeStruct((N, H), jnp.float32),
                   jax.ShapeDtypeStruct((N, H), jnp.float32)],
    )(xpre, aggp1, cntp, w_u1a, w_u1b, b_u1, w_p2, b_p2)

    edge_pass2 = _make_edge_pass(False)
    (aggp2,) = edge_pass2(m2, dst, nbr, zeros32, zeros16, ones)

    out_all = pl.pallas_call(
        _stage_c_body,
        out_shape=jax.ShapeDtypeStruct((N, NUM_CLASSES), jnp.float32),
    )(x1, aggp2, cntp, w_u2a, w_u2b, b_u2, w_post, b_post,
      params["out_W"], params["out_bias"])

    final_gather = _make_final_gather()
    return final_gather(out_all, input_node_indices)


# speedup vs baseline: 8.8100x; 1.1993x over previous
import functools

import jax
import jax.numpy as jnp
from jax import lax
from jax.experimental import pallas as pl
from jax.experimental.pallas import tpu as pltpu
from jax.experimental.pallas import tpu_sc as plsc

N = 10000
E = 320000
D_FEAT = 128
H = 32
NUM_CLASSES = 16
B = 1024

NC = 2
NS = 16
NW = NC * NS
CH = 128
NCH = 80
EPW = NCH * CH
PADE = NW * EPW
NBUF = 4
NPAD = 10240
NPAD2 = 10016
RP_SC = NPAD // NS


def _fold_ffn(p):
    s = p["g"] / jnp.sqrt(p["v"] + 1e-3)
    t = p["b"] - p["m"] * s
    W = p["W"] * s[:, None]
    b = t @ p["W"] + p["bias"]
    return W, b


def _stage_a_body(nf, w_pre, b_pre, w_p1, b_p1, xpre_o, m1_o):
    x = jax.nn.gelu(jnp.dot(nf[...], w_pre[...],
                            preferred_element_type=jnp.float32, precision=lax.Precision.HIGHEST) + b_pre[...])
    xpre_o[...] = x
    m1_o[0:N] = jax.nn.gelu(jnp.dot(x, w_p1[...],
                                    preferred_element_type=jnp.float32, precision=lax.Precision.HIGHEST) + b_p1[...])
    m1_o[N:NPAD2] = jnp.zeros((NPAD2 - N, H), jnp.float32)


def _stage_b_body(xpre, aggp, cntp, w_u1a, w_u1b, b_u1, w_p2, b_p2,
                  x1_o, m2_o):
    cnt = cntp[0, :N, 0:1] + cntp[1, :N, 0:1]
    inv = 1.0 / jnp.maximum(cnt, 1.0)
    agg = (aggp[0, :N] + aggp[1, :N]) * inv
    x = xpre[...]
    h = jax.nn.gelu(jnp.dot(x, w_u1a[...], preferred_element_type=jnp.float32, precision=lax.Precision.HIGHEST)
                    + jnp.dot(agg, w_u1b[...], preferred_element_type=jnp.float32, precision=lax.Precision.HIGHEST)
                    + b_u1[...])
    x1 = h + x
    x1_o[...] = x1
    m2_o[0:N] = jax.nn.gelu(jnp.dot(x1, w_p2[...],
                                    preferred_element_type=jnp.float32, precision=lax.Precision.HIGHEST) + b_p2[...])
    m2_o[N:NPAD2] = jnp.zeros((NPAD2 - N, H), jnp.float32)


def _stage_c_body(x1, aggp, cntp, w_u2a, w_u2b, b_u2, w_post, b_post,
                  w_out, b_out, out_o):
    cnt = cntp[0, :N, 0:1] + cntp[1, :N, 0:1]
    inv = 1.0 / jnp.maximum(cnt, 1.0)
    agg = (aggp[0, :N] + aggp[1, :N]) * inv
    x = x1[...]
    h = jax.nn.gelu(jnp.dot(x, w_u2a[...], preferred_element_type=jnp.float32, precision=lax.Precision.HIGHEST)
                    + jnp.dot(agg, w_u2b[...], preferred_element_type=jnp.float32, precision=lax.Precision.HIGHEST)
                    + b_u2[...])
    x2 = h + x
    xp = jax.nn.gelu(jnp.dot(x2, w_post[...],
                             preferred_element_type=jnp.float32, precision=lax.Precision.HIGHEST) + b_post[...])
    out_o[...] = jnp.dot(xp, w_out[...],
                         preferred_element_type=jnp.float32, precision=lax.Precision.HIGHEST) + b_out[...]


def _edge_pass_body(with_counts, m_hbm, dst3_hbm, nbr3_hbm, zeros32_hbm,
                    zeros16_hbm, ones_hbm, agg_out, cnt_out,
                    nbr_all, dst_all, rows, ones_v, agg_sh, cnt_sh, gsems):
    cid = lax.axis_index("c")
    sid = lax.axis_index("s")
    wid = cid * NS + sid

    rbase = sid * RP_SC
    pltpu.sync_copy(zeros32_hbm.at[pl.ds(rbase, RP_SC)],
                    agg_sh.at[pl.ds(rbase, RP_SC)])
    if with_counts:
        pltpu.sync_copy(zeros16_hbm.at[pl.ds(rbase, RP_SC)],
                        cnt_sh.at[pl.ds(rbase, RP_SC)])
        pltpu.sync_copy(ones_hbm, ones_v)
    pltpu.sync_copy(nbr3_hbm.at[wid], nbr_all)
    pltpu.sync_copy(dst3_hbm.at[wid], dst_all)
    plsc.subcore_barrier()

    for b in range(NBUF):
        pltpu.async_copy(m_hbm.at[nbr_all.at[b]], rows[b], gsems[b])

    def slot(j, b):
        pltpu.make_async_copy(m_hbm.at[nbr_all.at[j]], rows[b], gsems[b]).wait()
        pltpu.sync_copy(rows[b], agg_sh.at[dst_all.at[j]], add=True)
        if with_counts:
            pltpu.sync_copy(ones_v, cnt_sh.at[dst_all.at[j]], add=True)

    def outer(o, _):
        for b in range(NBUF):
            j = o * NBUF + b
            slot(j, b)
            pltpu.async_copy(m_hbm.at[nbr_all.at[j + NBUF]], rows[b], gsems[b])
        return 0

    lax.fori_loop(0, (NCH - NBUF) // NBUF, outer, 0)
    for b in range(NBUF):
        slot(NCH - NBUF + b, b)

    plsc.subcore_barrier()

    pltpu.sync_copy(agg_sh.at[pl.ds(rbase, RP_SC)],
                    agg_out.at[cid, pl.ds(rbase, RP_SC)])
    if with_counts:
        pltpu.sync_copy(cnt_sh.at[pl.ds(rbase, RP_SC)],
                        cnt_out.at[cid, pl.ds(rbase, RP_SC)])


def _make_edge_pass(with_counts):
    mesh = plsc.VectorSubcoreMesh(core_axis_name="c", subcore_axis_name="s")
    out_type = [jax.ShapeDtypeStruct((NC, NPAD, H), jnp.float32)]
    if with_counts:
        out_type.append(jax.ShapeDtypeStruct((NC, NPAD, 16), jnp.float32))
    scratch = [
        pltpu.VMEM((NCH, CH), jnp.int32),
        pltpu.VMEM((NCH, CH), jnp.int32),
        [pltpu.VMEM((CH, H), jnp.float32) for _ in range(NBUF)],
        pltpu.VMEM((CH, 16), jnp.float32),
        pltpu.VMEM_SHARED((NPAD, H), jnp.float32),
        pltpu.VMEM_SHARED((NPAD, 16), jnp.float32),
        [pltpu.SemaphoreType.DMA for _ in range(NBUF)],
    ]

    if with_counts:
        def body(m, d, nb, z32, z16, on, agg_o, cnt_o, *s):
            _edge_pass_body(True, m, d, nb, z32, z16, on, agg_o, cnt_o, *s)
    else:
        def body(m, d, nb, z32, z16, on, agg_o, *s):
            _edge_pass_body(False, m, d, nb, z32, z16, on, agg_o, None, *s)

    return pl.kernel(body, out_type=out_type, mesh=mesh, scratch_types=scratch,
                     compiler_params=pltpu.CompilerParams(use_tc_tiling_on_sc=False))


BPW = B // NW


def _final_gather_body(table_hbm, idx_hbm, out_hbm, idx_v, rows_v):
    wid = lax.axis_index("c") * NS + lax.axis_index("s")
    base = pl.multiple_of(wid * BPW, 8)
    pltpu.sync_copy(idx_hbm.at[pl.ds(base, BPW)], idx_v)
    pltpu.sync_copy(table_hbm.at[idx_v], rows_v)
    pltpu.sync_copy(rows_v, out_hbm.at[pl.ds(base, BPW)])


def _make_final_gather():
    mesh = plsc.VectorSubcoreMesh(core_axis_name="c", subcore_axis_name="s")
    return pl.kernel(
        _final_gather_body,
        out_type=jax.ShapeDtypeStruct((B, NUM_CLASSES), jnp.float32),
        mesh=mesh,
        scratch_types=[
            pltpu.VMEM((BPW,), jnp.int32),
            pltpu.VMEM((BPW, NUM_CLASSES), jnp.float32),
        ],
        compiler_params=pltpu.CompilerParams(use_tc_tiling_on_sc=False),
    )


def kernel(input_node_indices, node_features, edge_index, params):
    dst = edge_index[0]
    nbr = edge_index[1]

    w_pre, b_pre = _fold_ffn(params["pre"])
    w_p1, b_p1 = _fold_ffn(params["c1_prep"])
    w_u1, b_u1 = _fold_ffn(params["c1_upd"])
    w_p2, b_p2 = _fold_ffn(params["c2_prep"])
    w_u2, b_u2 = _fold_ffn(params["c2_upd"])
    w_post, b_post = _fold_ffn(params["post"])
    w_u1a, w_u1b = w_u1[:H], w_u1[H:]
    w_u2a, w_u2b = w_u2[:H], w_u2[H:]

    zeros32 = jnp.zeros((NPAD, H), jnp.float32)
    zeros16 = jnp.zeros((NPAD, 16), jnp.float32)
    ones = jnp.ones((CH, 16), jnp.float32)

    pad = jnp.full((PADE - E,), N, jnp.int32)
    nbr_p = jnp.concatenate([nbr, pad]).reshape(NW, NCH, CH)
    dst_p = jnp.concatenate([dst, jnp.full((PADE - E,), NPAD - 8, jnp.int32)]
                            ).reshape(NW, NCH, CH)

    xpre, m1 = pl.pallas_call(
        _stage_a_body,
        out_shape=[jax.ShapeDtypeStruct((N, H), jnp.float32),
                   jax.ShapeDtypeStruct((NPAD2, H), jnp.float32)],
    )(node_features, w_pre, b_pre, w_p1, b_p1)

    edge_pass1 = _make_edge_pass(True)
    aggp1, cntp = edge_pass1(m1, dst_p, nbr_p, zeros32, zeros16, ones)

    x1, m2 = pl.pallas_call(
        _stage_b_body,
        out_shape=[jax.ShapeDtypeStruct((N, H), jnp.float32),
                   jax.ShapeDtypeStruct((NPAD2, H), jnp.float32)],
    )(xpre, aggp1, cntp, w_u1a, w_u1b, b_u1, w_p2, b_p2)

    edge_pass2 = _make_edge_pass(False)
    (aggp2,) = edge_pass2(m2, dst_p, nbr_p, zeros32, zeros16, ones)

    out_all = pl.pallas_call(
        _stage_c_body,
        out_shape=jax.ShapeDtypeStruct((N, NUM_CLASSES), jnp.float32),
    )(x1, aggp2, cntp, w_u2a, w_u2b, b_u2, w_post, b_post,
      params["out_W"], params["out_bias"])

    final_gather = _make_final_gather()
    return final_gather(out_all, input_node_indices)


# speedup vs baseline: 16.2787x; 1.8478x over previous
import functools

import jax
import jax.numpy as jnp
from jax import lax
from jax.experimental import pallas as pl
from jax.experimental.pallas import tpu as pltpu
from jax.experimental.pallas import tpu_sc as plsc

N = 10000
E = 320000
D_FEAT = 128
H = 32
NUM_CLASSES = 16
B = 1024

NC = 2
NS = 16
NW = NC * NS
CH = 128
NCH = 80
EPW = NCH * CH
PADE = NW * EPW
NBUF = 4
NPAD = 10240
NPAD2 = 10016
RP_SC = NPAD // NS


def _fold(g, b, m, v, W, bias):
    s = g[...] / jnp.sqrt(v[...] + 1e-3)
    t = b[...] - m[...] * s
    Wf = W[...] * s[:, None]
    bf = jnp.dot(t.reshape(1, -1), W[...],
                 preferred_element_type=jnp.float32) + bias[...]
    return Wf, bf


def _p(params, name):
    p = params[name]
    return (p["g"], p["b"], p["m"], p["v"], p["W"], p["bias"])


def _stage_a_body(nf, *refs):
    (pre6, p16), (xpre_o, m1_o) = (refs[0:6], refs[6:12]), refs[12:]
    w_pre, b_pre = _fold(*pre6)
    w_p1, b_p1 = _fold(*p16)
    x = jax.nn.gelu(jnp.dot(nf[...], w_pre,
                            preferred_element_type=jnp.float32) + b_pre)
    xpre_o[...] = x
    m1_o[0:N] = jax.nn.gelu(jnp.dot(x, w_p1,
                                    preferred_element_type=jnp.float32) + b_p1)
    m1_o[N:NPAD2] = jnp.zeros((NPAD2 - N, H), jnp.float32)


def _stage_b_body(xpre, aggp, cntp, *refs):
    (u16, p26), (x1_o, m2_o) = (refs[0:6], refs[6:12]), refs[12:]
    w_u1, b_u1 = _fold(*u16)
    w_p2, b_p2 = _fold(*p26)
    cnt = cntp[0, :N, 0:1] + cntp[1, :N, 0:1]
    inv = 1.0 / jnp.maximum(cnt, 1.0)
    agg = (aggp[0, :N] + aggp[1, :N]) * inv
    x = xpre[...]
    h = jax.nn.gelu(jnp.dot(x, w_u1[:H], preferred_element_type=jnp.float32)
                    + jnp.dot(agg, w_u1[H:], preferred_element_type=jnp.float32)
                    + b_u1)
    x1 = h + x
    x1_o[...] = x1
    m2_o[0:N] = jax.nn.gelu(jnp.dot(x1, w_p2,
                                    preferred_element_type=jnp.float32) + b_p2)
    m2_o[N:NPAD2] = jnp.zeros((NPAD2 - N, H), jnp.float32)


def _stage_c_body(x1, aggp, cntp, *refs):
    (u26, post6), (w_out, b_out), (out_o,) = (
        (refs[0:6], refs[6:12]), refs[12:14], refs[14:])
    w_u2, b_u2 = _fold(*u26)
    w_post, b_post = _fold(*post6)
    cnt = cntp[0, :N, 0:1] + cntp[1, :N, 0:1]
    inv = 1.0 / jnp.maximum(cnt, 1.0)
    agg = (aggp[0, :N] + aggp[1, :N]) * inv
    x = x1[...]
    h = jax.nn.gelu(jnp.dot(x, w_u2[:H], preferred_element_type=jnp.float32)
                    + jnp.dot(agg, w_u2[H:], preferred_element_type=jnp.float32)
                    + b_u2)
    x2 = h + x
    xp = jax.nn.gelu(jnp.dot(x2, w_post,
                             preferred_element_type=jnp.float32) + b_post)
    out_o[...] = jnp.dot(xp, w_out[...],
                         preferred_element_type=jnp.float32) + b_out[...]


def _edge_pass_body(with_counts, m_hbm, dst3_hbm, nbr3_hbm, zeros32_hbm,
                    zeros16_hbm, ones_hbm, agg_out, cnt_out,
                    nbr_all, dst_all, rows, ones_v, agg_sh, cnt_sh, gsems):
    cid = lax.axis_index("c")
    sid = lax.axis_index("s")
    wid = cid * NS + sid

    rbase = sid * RP_SC
    pltpu.sync_copy(zeros32_hbm.at[pl.ds(rbase, RP_SC)],
                    agg_sh.at[pl.ds(rbase, RP_SC)])
    if with_counts:
        pltpu.sync_copy(zeros16_hbm.at[pl.ds(rbase, RP_SC)],
                        cnt_sh.at[pl.ds(rbase, RP_SC)])
        pltpu.sync_copy(ones_hbm, ones_v)
    pltpu.sync_copy(nbr3_hbm.at[wid], nbr_all)
    pltpu.sync_copy(dst3_hbm.at[wid], dst_all)
    plsc.subcore_barrier()

    for b in range(NBUF):
        pltpu.async_copy(m_hbm.at[nbr_all.at[b]], rows[b], gsems[b])

    def slot(j, b):
        pltpu.make_async_copy(m_hbm.at[nbr_all.at[j]], rows[b], gsems[b]).wait()
        pltpu.sync_copy(rows[b], agg_sh.at[dst_all.at[j]], add=True)
        if with_counts:
            pltpu.sync_copy(ones_v, cnt_sh.at[dst_all.at[j]], add=True)

    def outer(o, _):
        for b in range(NBUF):
            j = o * NBUF + b
            slot(j, b)
            pltpu.async_copy(m_hbm.at[nbr_all.at[j + NBUF]], rows[b], gsems[b])
        return 0

    lax.fori_loop(0, (NCH - NBUF) // NBUF, outer, 0)
    for b in range(NBUF):
        slot(NCH - NBUF + b, b)

    plsc.subcore_barrier()

    pltpu.sync_copy(agg_sh.at[pl.ds(rbase, RP_SC)],
                    agg_out.at[cid, pl.ds(rbase, RP_SC)])
    if with_counts:
        pltpu.sync_copy(cnt_sh.at[pl.ds(rbase, RP_SC)],
                        cnt_out.at[cid, pl.ds(rbase, RP_SC)])


def _make_edge_pass(with_counts):
    mesh = plsc.VectorSubcoreMesh(core_axis_name="c", subcore_axis_name="s")
    out_type = [jax.ShapeDtypeStruct((NC, NPAD, H), jnp.float32)]
    if with_counts:
        out_type.append(jax.ShapeDtypeStruct((NC, NPAD, 16), jnp.float32))
    scratch = [
        pltpu.VMEM((NCH, CH), jnp.int32),
        pltpu.VMEM((NCH, CH), jnp.int32),
        [pltpu.VMEM((CH, H), jnp.float32) for _ in range(NBUF)],
        pltpu.VMEM((CH, 16), jnp.float32),
        pltpu.VMEM_SHARED((NPAD, H), jnp.float32),
        pltpu.VMEM_SHARED((NPAD, 16), jnp.float32),
        [pltpu.SemaphoreType.DMA for _ in range(NBUF)],
    ]

    if with_counts:
        def body(m, d, nb, z32, z16, on, agg_o, cnt_o, *s):
            _edge_pass_body(True, m, d, nb, z32, z16, on, agg_o, cnt_o, *s)
    else:
        def body(m, d, nb, z32, z16, on, agg_o, *s):
            _edge_pass_body(False, m, d, nb, z32, z16, on, agg_o, None, *s)

    return pl.kernel(body, out_type=out_type, mesh=mesh, scratch_types=scratch,
                     compiler_params=pltpu.CompilerParams(use_tc_tiling_on_sc=False))


BPW = B // NW


def _final_gather_body(table_hbm, idx_hbm, out_hbm, idx_v, rows_v):
    wid = lax.axis_index("c") * NS + lax.axis_index("s")
    base = pl.multiple_of(wid * BPW, 8)
    pltpu.sync_copy(idx_hbm.at[pl.ds(base, BPW)], idx_v)
    pltpu.sync_copy(table_hbm.at[idx_v], rows_v)
    pltpu.sync_copy(rows_v, out_hbm.at[pl.ds(base, BPW)])


def _make_final_gather():
    mesh = plsc.VectorSubcoreMesh(core_axis_name="c", subcore_axis_name="s")
    return pl.kernel(
        _final_gather_body,
        out_type=jax.ShapeDtypeStruct((B, NUM_CLASSES), jnp.float32),
        mesh=mesh,
        scratch_types=[
            pltpu.VMEM((BPW,), jnp.int32),
            pltpu.VMEM((BPW, NUM_CLASSES), jnp.float32),
        ],
        compiler_params=pltpu.CompilerParams(use_tc_tiling_on_sc=False),
    )


def kernel(input_node_indices, node_features, edge_index, params):
    dst = edge_index[0]
    nbr = edge_index[1]

    zeros32 = jnp.zeros((NPAD, H), jnp.float32)
    zeros16 = jnp.zeros((NPAD, 16), jnp.float32)
    ones = jnp.ones((CH, 16), jnp.float32)

    pad_i = jnp.arange(PADE - E, dtype=jnp.int32)
    nbr_p = jnp.concatenate([nbr, N + (pad_i % (NPAD2 - N))]).reshape(NW, NCH, CH)
    dst_p = jnp.concatenate([dst, N + (pad_i % (NPAD - N))]).reshape(NW, NCH, CH)

    xpre, m1 = pl.pallas_call(
        _stage_a_body,
        out_shape=[jax.ShapeDtypeStruct((N, H), jnp.float32),
                   jax.ShapeDtypeStruct((NPAD2, H), jnp.float32)],
    )(node_features, *_p(params, "pre"), *_p(params, "c1_prep"))

    edge_pass1 = _make_edge_pass(True)
    aggp1, cntp = edge_pass1(m1, dst_p, nbr_p, zeros32, zeros16, ones)

    x1, m2 = pl.pallas_call(
        _stage_b_body,
        out_shape=[jax.ShapeDtypeStruct((N, H), jnp.float32),
                   jax.ShapeDtypeStruct((NPAD2, H), jnp.float32)],
    )(xpre, aggp1, cntp, *_p(params, "c1_upd"), *_p(params, "c2_prep"))

    edge_pass2 = _make_edge_pass(False)
    (aggp2,) = edge_pass2(m2, dst_p, nbr_p, zeros32, zeros16, ones)

    out_all = pl.pallas_call(
        _stage_c_body,
        out_shape=jax.ShapeDtypeStruct((N, NUM_CLASSES), jnp.float32),
    )(x1, aggp2, cntp, *_p(params, "c2_upd"), *_p(params, "post"),
      params["out_W"], params["out_bias"])

    final_gather = _make_final_gather()
    return final_gather(out_all, input_node_indices)


# speedup vs baseline: 16.2852x; 1.0004x over previous
import functools

import jax
import jax.numpy as jnp
from jax import lax
from jax.experimental import pallas as pl
from jax.experimental.pallas import tpu as pltpu
from jax.experimental.pallas import tpu_sc as plsc

N = 10000
E = 320000
D_FEAT = 128
H = 32
NUM_CLASSES = 16
B = 1024

NC = 2
NS = 16
NW = NC * NS
CH = 128
NCH = 80
EPW = NCH * CH
PADE = NW * EPW
RING = 8
DEPTH = 4
NPAD = 10240
NPAD2 = 10016
RP_SC = NPAD // NS


def _fold(g, b, m, v, W, bias):
    s = g[...] / jnp.sqrt(v[...] + 1e-3)
    t = b[...] - m[...] * s
    Wf = W[...] * s[:, None]
    bf = jnp.dot(t.reshape(1, -1), W[...],
                 preferred_element_type=jnp.float32) + bias[...]
    return Wf, bf


def _p(params, name):
    p = params[name]
    return (p["g"], p["b"], p["m"], p["v"], p["W"], p["bias"])


def _stage_a_body(nf, *refs):
    (pre6, p16), (xpre_o, m1_o) = (refs[0:6], refs[6:12]), refs[12:]
    w_pre, b_pre = _fold(*pre6)
    w_p1, b_p1 = _fold(*p16)
    x = jax.nn.gelu(jnp.dot(nf[...], w_pre,
                            preferred_element_type=jnp.float32) + b_pre)
    xpre_o[...] = x
    m1_o[0:N] = jax.nn.gelu(jnp.dot(x, w_p1,
                                    preferred_element_type=jnp.float32) + b_p1)
    m1_o[N:NPAD2] = jnp.zeros((NPAD2 - N, H), jnp.float32)


def _stage_b_body(xpre, aggp, cntp, *refs):
    (u16, p26), (x1_o, m2_o) = (refs[0:6], refs[6:12]), refs[12:]
    w_u1, b_u1 = _fold(*u16)
    w_p2, b_p2 = _fold(*p26)
    cnt = cntp[0, :N, 0:1] + cntp[1, :N, 0:1]
    inv = 1.0 / jnp.maximum(cnt, 1.0)
    agg = (aggp[0, :N] + aggp[1, :N]) * inv
    x = xpre[...]
    h = jax.nn.gelu(jnp.dot(x, w_u1[:H], preferred_element_type=jnp.float32)
                    + jnp.dot(agg, w_u1[H:], preferred_element_type=jnp.float32)
                    + b_u1)
    x1 = h + x
    x1_o[...] = x1
    m2_o[0:N] = jax.nn.gelu(jnp.dot(x1, w_p2,
                                    preferred_element_type=jnp.float32) + b_p2)
    m2_o[N:NPAD2] = jnp.zeros((NPAD2 - N, H), jnp.float32)


def _stage_c_body(x1, aggp, cntp, *refs):
    (u26, post6), (w_out, b_out), (out_o,) = (
        (refs[0:6], refs[6:12]), refs[12:14], refs[14:])
    w_u2, b_u2 = _fold(*u26)
    w_post, b_post = _fold(*post6)
    cnt = cntp[0, :N, 0:1] + cntp[1, :N, 0:1]
    inv = 1.0 / jnp.maximum(cnt, 1.0)
    agg = (aggp[0, :N] + aggp[1, :N]) * inv
    x = x1[...]
    h = jax.nn.gelu(jnp.dot(x, w_u2[:H], preferred_element_type=jnp.float32)
                    + jnp.dot(agg, w_u2[H:], preferred_element_type=jnp.float32)
                    + b_u2)
    x2 = h + x
    xp = jax.nn.gelu(jnp.dot(x2, w_post,
                             preferred_element_type=jnp.float32) + b_post)
    out_o[...] = jnp.dot(xp, w_out[...],
                         preferred_element_type=jnp.float32) + b_out[...]


def _edge_pass_body(with_counts, m_hbm, dst3_hbm, nbr3_hbm, zeros32_hbm,
                    zeros16_hbm, ones_hbm, agg_out, cnt_out,
                    nbr_all, dst_all, rows, ones_v, agg_sh, cnt_sh, sems):
    cid = lax.axis_index("c")
    sid = lax.axis_index("s")
    wid = cid * NS + sid

    rbase = sid * RP_SC
    pltpu.sync_copy(zeros32_hbm.at[pl.ds(rbase, RP_SC)],
                    agg_sh.at[pl.ds(rbase, RP_SC)])
    if with_counts:
        pltpu.sync_copy(zeros16_hbm.at[pl.ds(rbase, RP_SC)],
                        cnt_sh.at[pl.ds(rbase, RP_SC)])
        pltpu.sync_copy(ones_hbm, ones_v)
    pltpu.sync_copy(nbr3_hbm.at[wid], nbr_all)
    pltpu.sync_copy(dst3_hbm.at[wid], dst_all)
    plsc.subcore_barrier()

    gsems, ssems, csem = sems[0:RING], sems[RING:2 * RING], sems[2 * RING]

    def gather(j, b):
        pltpu.async_copy(m_hbm.at[nbr_all.at[j]], rows[b], gsems[b])

    def gwait(j, b):
        pltpu.make_async_copy(m_hbm.at[nbr_all.at[j]], rows[b], gsems[b]).wait()

    def scatter(j, b):
        pltpu.async_copy(rows[b], agg_sh.at[dst_all.at[j]], ssems[b], add=True)
        if with_counts:
            pltpu.async_copy(ones_v, cnt_sh.at[dst_all.at[j]], csem, add=True)

    def swait(b):
        pltpu.make_async_copy(rows[b], agg_sh.at[dst_all.at[0]], ssems[b]).wait()

    for b in range(DEPTH):
        gather(b, b)
    for b in range(DEPTH):
        gwait(b, b)
        scatter(b, b)
        gather(b + DEPTH, b + DEPTH)

    def outer(o, _):
        for b0 in range(RING):
            j = DEPTH + o * RING + b0
            bb = (DEPTH + b0) % RING
            nb = (bb + DEPTH) % RING
            gwait(j, bb)
            scatter(j, bb)
            swait(nb)
            gather(j + DEPTH, nb)
        return 0

    lax.fori_loop(0, (NCH - 2 * DEPTH) // RING, outer, 0)
    for b0 in range(DEPTH):
        j = NCH - DEPTH + b0
        bb = j % RING
        gwait(j, bb)
        scatter(j, bb)
    for b in range(RING):
        swait(b)
    if with_counts:

        def cdrain(_, __):
            pltpu.make_async_copy(ones_v, cnt_sh.at[dst_all.at[0]], csem).wait()
            return 0

        lax.fori_loop(0, NCH, cdrain, 0)

    plsc.subcore_barrier()

    pltpu.sync_copy(agg_sh.at[pl.ds(rbase, RP_SC)],
                    agg_out.at[cid, pl.ds(rbase, RP_SC)])
    if with_counts:
        pltpu.sync_copy(cnt_sh.at[pl.ds(rbase, RP_SC)],
                        cnt_out.at[cid, pl.ds(rbase, RP_SC)])


def _make_edge_pass(with_counts):
    mesh = plsc.VectorSubcoreMesh(core_axis_name="c", subcore_axis_name="s")
    out_type = [jax.ShapeDtypeStruct((NC, NPAD, H), jnp.float32)]
    if with_counts:
        out_type.append(jax.ShapeDtypeStruct((NC, NPAD, 16), jnp.float32))
    scratch = [
        pltpu.VMEM((NCH, CH), jnp.int32),
        pltpu.VMEM((NCH, CH), jnp.int32),
        [pltpu.VMEM((CH, H), jnp.float32) for _ in range(RING)],
        pltpu.VMEM((CH, 16), jnp.float32),
        pltpu.VMEM_SHARED((NPAD, H), jnp.float32),
        pltpu.VMEM_SHARED((NPAD, 16), jnp.float32),
        [pltpu.SemaphoreType.DMA for _ in range(2 * RING + 1)],
    ]

    if with_counts:
        def body(m, d, nb, z32, z16, on, agg_o, cnt_o, *s):
            _edge_pass_body(True, m, d, nb, z32, z16, on, agg_o, cnt_o, *s)
    else:
        def body(m, d, nb, z32, z16, on, agg_o, *s):
            _edge_pass_body(False, m, d, nb, z32, z16, on, agg_o, None, *s)

    return pl.kernel(body, out_type=out_type, mesh=mesh, scratch_types=scratch,
                     compiler_params=pltpu.CompilerParams(use_tc_tiling_on_sc=False))


BPW = B // NW


def _final_gather_body(table_hbm, idx_hbm, out_hbm, idx_v, rows_v):
    wid = lax.axis_index("c") * NS + lax.axis_index("s")
    base = pl.multiple_of(wid * BPW, 8)
    pltpu.sync_copy(idx_hbm.at[pl.ds(base, BPW)], idx_v)
    pltpu.sync_copy(table_hbm.at[idx_v], rows_v)
    pltpu.sync_copy(rows_v, out_hbm.at[pl.ds(base, BPW)])


def _make_final_gather():
    mesh = plsc.VectorSubcoreMesh(core_axis_name="c", subcore_axis_name="s")
    return pl.kernel(
        _final_gather_body,
        out_type=jax.ShapeDtypeStruct((B, NUM_CLASSES), jnp.float32),
        mesh=mesh,
        scratch_types=[
            pltpu.VMEM((BPW,), jnp.int32),
            pltpu.VMEM((BPW, NUM_CLASSES), jnp.float32),
        ],
        compiler_params=pltpu.CompilerParams(use_tc_tiling_on_sc=False),
    )


def kernel(input_node_indices, node_features, edge_index, params):
    dst = edge_index[0]
    nbr = edge_index[1]

    zeros32 = jnp.zeros((NPAD, H), jnp.float32)
    zeros16 = jnp.zeros((NPAD, 16), jnp.float32)
    ones = jnp.ones((CH, 16), jnp.float32)

    pad_i = jnp.arange(PADE - E, dtype=jnp.int32)
    nbr_p = jnp.concatenate([nbr, N + (pad_i % (NPAD2 - N))]).reshape(NW, NCH, CH)
    dst_p = jnp.concatenate([dst, N + (pad_i % (NPAD - N))]).reshape(NW, NCH, CH)

    xpre, m1 = pl.pallas_call(
        _stage_a_body,
        out_shape=[jax.ShapeDtypeStruct((N, H), jnp.float32),
                   jax.ShapeDtypeStruct((NPAD2, H), jnp.float32)],
    )(node_features, *_p(params, "pre"), *_p(params, "c1_prep"))

    edge_pass1 = _make_edge_pass(True)
    aggp1, cntp = edge_pass1(m1, dst_p, nbr_p, zeros32, zeros16, ones)

    x1, m2 = pl.pallas_call(
        _stage_b_body,
        out_shape=[jax.ShapeDtypeStruct((N, H), jnp.float32),
                   jax.ShapeDtypeStruct((NPAD2, H), jnp.float32)],
    )(xpre, aggp1, cntp, *_p(params, "c1_upd"), *_p(params, "c2_prep"))

    edge_pass2 = _make_edge_pass(False)
    (aggp2,) = edge_pass2(m2, dst_p, nbr_p, zeros32, zeros16, ones)

    out_all = pl.pallas_call(
        _stage_c_body,
        out_shape=jax.ShapeDtypeStruct((N, NUM_CLASSES), jnp.float32),
    )(x1, aggp2, cntp, *_p(params, "c2_upd"), *_p(params, "post"),
      params["out_W"], params["out_bias"])

    final_gather = _make_final_gather()
    return final_gather(out_all, input_node_indices)


# speedup vs baseline: 19.6005x; 1.2036x over previous
import functools

import jax
import jax.numpy as jnp
from jax import lax
from jax.experimental import pallas as pl
from jax.experimental.pallas import tpu as pltpu
from jax.experimental.pallas import tpu_sc as plsc

N = 10000
E = 320000
D_FEAT = 128
H = 32
NUM_CLASSES = 16
B = 1024

NC = 2
NS = 16
NW = NC * NS
CH = 128
NCH = 80
EPW = NCH * CH
PADE = NW * EPW
RING = 8
DEPTH = 4
NPAD = 10240
NPAD2 = NPAD
RP_SC = NPAD // NS


def _fold(g, b, m, v, W, bias):
    s = g[...] / jnp.sqrt(v[...] + 1e-3)
    t = b[...] - m[...] * s
    Wf = W[...] * s[:, None]
    bf = jnp.dot(t.reshape(1, -1), W[...],
                 preferred_element_type=jnp.float32) + bias[...]
    return Wf, bf


def _p(params, name):
    p = params[name]
    return (p["g"], p["b"], p["m"], p["v"], p["W"], p["bias"])


def _stage_a_body(nf, *refs):
    (pre6, p16), (xpre_o, m1_o) = (refs[0:6], refs[6:12]), refs[12:]
    w_pre, b_pre = _fold(*pre6)
    w_p1, b_p1 = _fold(*p16)
    x = jax.nn.gelu(jnp.dot(nf[...], w_pre,
                            preferred_element_type=jnp.float32) + b_pre)
    xpre_o[...] = x
    m1_o[0:N] = jax.nn.gelu(jnp.dot(x, w_p1,
                                    preferred_element_type=jnp.float32) + b_p1)
    m1_o[N:NPAD2] = jnp.zeros((NPAD2 - N, H), jnp.float32)


def _stage_b_body(xpre, aggp, cntp, *refs):
    (u16, p26), (x1_o, m2_o) = (refs[0:6], refs[6:12]), refs[12:]
    w_u1, b_u1 = _fold(*u16)
    w_p2, b_p2 = _fold(*p26)
    cnt = cntp[0, :N, 0:1] + cntp[1, :N, 0:1]
    inv = 1.0 / jnp.maximum(cnt, 1.0)
    agg = (aggp[0, :N] + aggp[1, :N]) * inv
    x = xpre[...]
    h = jax.nn.gelu(jnp.dot(x, w_u1[:H], preferred_element_type=jnp.float32)
                    + jnp.dot(agg, w_u1[H:], preferred_element_type=jnp.float32)
                    + b_u1)
    x1 = h + x
    x1_o[...] = x1
    m2_o[0:N] = jax.nn.gelu(jnp.dot(x1, w_p2,
                                    preferred_element_type=jnp.float32) + b_p2)
    m2_o[N:NPAD2] = jnp.zeros((NPAD2 - N, H), jnp.float32)


def _stage_c_body(x1, aggp, cntp, *refs):
    (u26, post6), (w_out, b_out), (out_o,) = (
        (refs[0:6], refs[6:12]), refs[12:14], refs[14:])
    w_u2, b_u2 = _fold(*u26)
    w_post, b_post = _fold(*post6)
    cnt = cntp[0, :N, 0:1] + cntp[1, :N, 0:1]
    inv = 1.0 / jnp.maximum(cnt, 1.0)
    agg = (aggp[0, :N] + aggp[1, :N]) * inv
    x = x1[...]
    h = jax.nn.gelu(jnp.dot(x, w_u2[:H], preferred_element_type=jnp.float32)
                    + jnp.dot(agg, w_u2[H:], preferred_element_type=jnp.float32)
                    + b_u2)
    x2 = h + x
    xp = jax.nn.gelu(jnp.dot(x2, w_post,
                             preferred_element_type=jnp.float32) + b_post)
    out_o[...] = jnp.dot(xp, w_out[...],
                         preferred_element_type=jnp.float32) + b_out[...]


def _edge_pass_body(with_counts, m_hbm, ei_hbm, zeros32_hbm,
                    agg_out, cnt_out,
                    nbr_all, dst_all, rows, m_sh, agg_sh, sems,
                    hist_v, acc_v, tmp_v, cnt16_v, cnt_part):
    cid = lax.axis_index("c")
    sid = lax.axis_index("s")
    wid = cid * NS + sid

    rbase = sid * RP_SC
    pltpu.sync_copy(zeros32_hbm.at[pl.ds(rbase, RP_SC)],
                    agg_sh.at[pl.ds(rbase, RP_SC)])
    if with_counts:
        def zh(i, _):
            hist_v[pl.ds(i * 16, 16)] = jnp.zeros((16,), jnp.float32)
            return 0

        lax.fori_loop(0, NPAD // 16, zh, 0)
    pltpu.sync_copy(ei_hbm.at[1, wid], nbr_all)
    pltpu.sync_copy(ei_hbm.at[0, wid], dst_all)
    pltpu.sync_copy(m_hbm.at[pl.ds(rbase, RP_SC)], m_sh.at[pl.ds(rbase, RP_SC)])
    plsc.subcore_barrier()

    gsems, ssems = sems[0:RING], sems[RING:2 * RING]
    ones16 = jnp.ones((16,), jnp.float32)

    def hist_chunk(j):
        if with_counts:
            for k in range(CH // 16):
                plsc.addupdate_scatter(
                    hist_v, [dst_all[j, pl.ds(k * 16, 16)]], ones16)

    def gather(j, b):
        pltpu.async_copy(m_sh.at[nbr_all.at[j]], rows[b], gsems[b])

    def gwait(j, b):
        pltpu.make_async_copy(m_sh.at[nbr_all.at[j]], rows[b], gsems[b]).wait()

    def scatter(j, b):
        pltpu.async_copy(rows[b], agg_sh.at[dst_all.at[j]], ssems[b], add=True)

    def swait(b):
        pltpu.make_async_copy(rows[b], agg_sh.at[dst_all.at[0]], ssems[b]).wait()

    for b in range(DEPTH):
        gather(b, b)
    for b in range(DEPTH):
        gwait(b, b)
        scatter(b, b)
        gather(b + DEPTH, b + DEPTH)
        hist_chunk(b)

    def outer(o, _):
        for b0 in range(RING):
            j = DEPTH + o * RING + b0
            bb = (DEPTH + b0) % RING
            nb = (bb + DEPTH) % RING
            gwait(j, bb)
            scatter(j, bb)
            swait(nb)
            gather(j + DEPTH, nb)
            hist_chunk(j)
        return 0

    lax.fori_loop(0, (NCH - 2 * DEPTH) // RING, outer, 0)
    for b0 in range(DEPTH):
        j = NCH - DEPTH + b0
        bb = j % RING
        gwait(j, bb)
        scatter(j, bb)
        hist_chunk(j)
    for b in range(RING):
        swait(b)
    if with_counts:
        pltpu.sync_copy(hist_v, cnt_part.at[sid])

    plsc.subcore_barrier()

    pltpu.sync_copy(agg_sh.at[pl.ds(rbase, RP_SC)],
                    agg_out.at[cid, pl.ds(rbase, RP_SC)])
    if with_counts:
        pltpu.sync_copy(cnt_part.at[0, pl.ds(rbase, RP_SC)], acc_v)
        for k in range(1, NS):
            pltpu.sync_copy(cnt_part.at[k, pl.ds(rbase, RP_SC)], tmp_v)

            def addk(r, _):
                acc_v[pl.ds(r * 16, 16)] = (acc_v[pl.ds(r * 16, 16)]
                                            + tmp_v[pl.ds(r * 16, 16)])
                return 0

            lax.fori_loop(0, RP_SC // 16, addk, 0)

        def cwide(q, _):
            cv = acc_v[pl.ds(q * 16, 16)]
            for kk in range(16):
                r = q * 16 + kk
                cnt16_v[r, pl.ds(0, 16)] = jnp.full((16,), cv[kk], jnp.float32)
            return 0

        lax.fori_loop(0, RP_SC // 16, cwide, 0)
        pltpu.sync_copy(cnt16_v, cnt_out.at[cid, pl.ds(rbase, RP_SC)])


def _make_edge_pass(with_counts):
    mesh = plsc.VectorSubcoreMesh(core_axis_name="c", subcore_axis_name="s")
    out_type = [jax.ShapeDtypeStruct((NC, NPAD, H), jnp.float32)]
    if with_counts:
        out_type.append(jax.ShapeDtypeStruct((NC, NPAD, 16), jnp.float32))
    scratch = [
        pltpu.VMEM((NCH, CH), jnp.int32),
        pltpu.VMEM((NCH, CH), jnp.int32),
        [pltpu.VMEM((CH, H), jnp.float32) for _ in range(RING)],
        pltpu.VMEM_SHARED((NPAD, H), jnp.float32),
        pltpu.VMEM_SHARED((NPAD, H), jnp.float32),
        [pltpu.SemaphoreType.DMA for _ in range(2 * RING)],
        pltpu.VMEM((NPAD,) if with_counts else (16,), jnp.float32),
        pltpu.VMEM((RP_SC,), jnp.float32),
        pltpu.VMEM((RP_SC,), jnp.float32),
        pltpu.VMEM((RP_SC, 16) if with_counts else (8, 16),
                   jnp.float32),
        pltpu.VMEM_SHARED((NS, NPAD) if with_counts else (8, 8),
                          jnp.float32),
    ]

    if with_counts:
        def body(m, ei, z32, agg_o, cnt_o, *s):
            _edge_pass_body(True, m, ei, z32, agg_o, cnt_o, *s)
    else:
        def body(m, ei, z32, agg_o, *s):
            _edge_pass_body(False, m, ei, z32, agg_o, None, *s)

    return pl.kernel(body, out_type=out_type, mesh=mesh, scratch_types=scratch,
                     compiler_params=pltpu.CompilerParams(
                         use_tc_tiling_on_sc=False, needs_layout_passes=False))


BPW = B // NW


def _final_gather_body(table_hbm, idx_hbm, out_hbm, idx_v, rows_v):
    wid = lax.axis_index("c") * NS + lax.axis_index("s")
    base = pl.multiple_of(wid * BPW, 8)
    pltpu.sync_copy(idx_hbm.at[pl.ds(base, BPW)], idx_v)
    pltpu.sync_copy(table_hbm.at[idx_v], rows_v)
    pltpu.sync_copy(rows_v, out_hbm.at[pl.ds(base, BPW)])


def _make_final_gather():
    mesh = plsc.VectorSubcoreMesh(core_axis_name="c", subcore_axis_name="s")
    return pl.kernel(
        _final_gather_body,
        out_type=jax.ShapeDtypeStruct((B, NUM_CLASSES), jnp.float32),
        mesh=mesh,
        scratch_types=[
            pltpu.VMEM((BPW,), jnp.int32),
            pltpu.VMEM((BPW, NUM_CLASSES), jnp.float32),
        ],
        compiler_params=pltpu.CompilerParams(use_tc_tiling_on_sc=False),
    )


def kernel(input_node_indices, node_features, edge_index, params):
    zeros32 = jnp.zeros((NPAD, H), jnp.float32)

    pad_i = jnp.arange(PADE - E, dtype=jnp.int32)
    pad_rows = jnp.stack([N + (pad_i % (NPAD - N)), N + (pad_i % (NPAD2 - N))])
    ei_p = jnp.concatenate([edge_index, pad_rows], axis=1).reshape(2, NW, NCH, CH)

    xpre, m1 = pl.pallas_call(
        _stage_a_body,
        out_shape=[jax.ShapeDtypeStruct((N, H), jnp.float32),
                   jax.ShapeDtypeStruct((NPAD2, H), jnp.float32)],
    )(node_features, *_p(params, "pre"), *_p(params, "c1_prep"))

    edge_pass1 = _make_edge_pass(True)
    aggp1, cntp = edge_pass1(m1, ei_p, zeros32)

    x1, m2 = pl.pallas_call(
        _stage_b_body,
        out_shape=[jax.ShapeDtypeStruct((N, H), jnp.float32),
                   jax.ShapeDtypeStruct((NPAD2, H), jnp.float32)],
    )(xpre, aggp1, cntp, *_p(params, "c1_upd"), *_p(params, "c2_prep"))

    edge_pass2 = _make_edge_pass(False)
    (aggp2,) = edge_pass2(m2, ei_p, zeros32)

    out_all = pl.pallas_call(
        _stage_c_body,
        out_shape=jax.ShapeDtypeStruct((N, NUM_CLASSES), jnp.float32),
    )(x1, aggp2, cntp, *_p(params, "c2_upd"), *_p(params, "post"),
      params["out_W"], params["out_bias"])

    final_gather = _make_final_gather()
    return final_gather(out_all, input_node_indices)
